# baseline (speedup 1.0000x reference)
import functools

import jax
import jax.numpy as jnp
from jax import lax
from jax.experimental import pallas as pl
from jax.experimental.pallas import tpu as pltpu
from jax.experimental.pallas import tpu_sc as plsc

N = 10000
D = 128

_NC = 2
_NS = 16
_NW = _NC * _NS
_CHUNK = 128
_CPW = 3
_BPW = _CHUNK * _CPW
_B_PAD = _NW * _BPW


def _make_sc_gather():
    mesh = plsc.VectorSubcoreMesh(core_axis_name="c", subcore_axis_name="s")

    @functools.partial(
        pl.kernel,
        mesh=mesh,
        out_type=jax.ShapeDtypeStruct((_B_PAD, D), jnp.float32),
        scratch_types=[
            pltpu.VMEM((_BPW,), jnp.int32),
            pltpu.VMEM((_BPW, D), jnp.float32),
            pltpu.SemaphoreType.DMA,
        ],
    )
    def gather_kernel(table_hbm, idx_hbm, out_hbm, idx_v, rows_v, sem):
        wid = lax.axis_index("s") * _NC + lax.axis_index("c")
        base = wid * _BPW
        pltpu.sync_copy(idx_hbm.at[pl.ds(base, _BPW)], idx_v)
        copies = []
        for j in range(_CPW):
            copies.append(
                pltpu.async_copy(
                    table_hbm.at[idx_v.at[pl.ds(j * _CHUNK, _CHUNK)]],
                    rows_v.at[pl.ds(j * _CHUNK, _CHUNK)],
                    sem,
                )
            )
        for c in copies:
            c.wait()
        pltpu.sync_copy(rows_v, out_hbm.at[pl.ds(base, _BPW)])

    return gather_kernel


_sc_gather_cache = []


def _sc_gather(table, idx):
    if not _sc_gather_cache:
        _sc_gather_cache.append(_make_sc_gather())
    return _sc_gather_cache[0](table, idx)


def _prep_body(ue_ref, w1_ref, w2_ref, b2_ref, lw1_ref, lb1_ref, lw2_ref,
               lb2_ref, s1_ref, v_ref, c_ref):
    s1_ref[...] = jnp.dot(ue_ref[...], w1_ref[...],
                          preferred_element_type=jnp.float32)
    t = jnp.dot(lw1_ref[...], lw2_ref[...],
                preferred_element_type=jnp.float32)
    v_ref[...] = jnp.dot(w2_ref[...], t,
                         preferred_element_type=jnp.float32)
    c_ref[...] = (jnp.dot(b2_ref[...], t,
                          preferred_element_type=jnp.float32)
                  + jnp.dot(lb1_ref[...], lw2_ref[...],
                            preferred_element_type=jnp.float32)
                  + lb2_ref[...])


def _prep(user_emb, W1, W2, b2, lw1, lb1, lw2, lb2):
    return pl.pallas_call(
        _prep_body,
        out_shape=(
            jax.ShapeDtypeStruct((N, D), jnp.float32),
            jax.ShapeDtypeStruct((D, 1), jnp.float32),
            jax.ShapeDtypeStruct((1, 1), jnp.float32),
        ),
    )(user_emb, W1, W2, b2.reshape(1, D), lw1, lb1.reshape(1, 16), lw2,
      lb2.reshape(1, 1))


_BM = 200
_NBLK = N // _BM


def _pass_a_body(adj_ref, s1_ref, b1_ref, v_ref, out_ref):
    h = jnp.dot(adj_ref[...], s1_ref[...], preferred_element_type=jnp.float32)
    h = jnp.maximum(h + b1_ref[...], 0.0)
    out_ref[...] = jnp.sum(h * v_ref[...], axis=1, keepdims=True)


def _pass_a(adj, s1, b1, v_row):
    return pl.pallas_call(
        _pass_a_body,
        grid=(_NBLK,),
        in_specs=[
            pl.BlockSpec((_BM, N), lambda m: (m, 0)),
            pl.BlockSpec((N, D), lambda m: (0, 0)),
            pl.BlockSpec((1, D), lambda m: (0, 0)),
            pl.BlockSpec((1, D), lambda m: (0, 0)),
        ],
        out_specs=pl.BlockSpec((_BM, 1), lambda m: (m, 0)),
        out_shape=jax.ShapeDtypeStruct((N, 1), jnp.float32),
        compiler_params=pltpu.CompilerParams(
            dimension_semantics=("arbitrary",)),
    )(adj, s1, b1.reshape(1, D), v_row)


def _pass_b_body(adj_ref, hv_ref, c_ref, out_ref):
    out_ref[...] = (jnp.sum(adj_ref[...] * hv_ref[...], axis=1,
                            keepdims=True) + c_ref[0, 0])


def _pass_b(adj, hv_row, c):
    return pl.pallas_call(
        _pass_b_body,
        grid=(_NBLK,),
        in_specs=[
            pl.BlockSpec((_BM, N), lambda m: (m, 0)),
            pl.BlockSpec((1, N), lambda m: (0, 0)),
            pl.BlockSpec((1, 1), lambda m: (0, 0)),
        ],
        out_specs=pl.BlockSpec((_BM, 1), lambda m: (m, 0)),
        out_shape=jax.ShapeDtypeStruct((N, 1), jnp.float32),
        compiler_params=pltpu.CompilerParams(
            dimension_semantics=("arbitrary",)),
    )(adj, hv_row, c)


def kernel(features, adj, emb_table, W1, b1, W2, b2, lw1, lb1, lw2, lb2):
    idx = jnp.zeros((_B_PAD,), jnp.int32).at[:N].set(
        features.astype(jnp.int32))
    emb_pad = _sc_gather(emb_table, idx)
    user_emb = emb_pad[:N]
    s1, v, c = _prep(user_emb, W1, W2, b2, lw1, lb1, lw2, lb2)
    hv = _pass_a(adj, s1, b1, v.reshape(1, D))
    x = _pass_b(adj, hv.reshape(1, N), c)
    return (x, user_emb)


# speedup vs baseline: 1.0027x; 1.0027x over previous
import functools

import jax
import jax.numpy as jnp
from jax import lax
from jax.experimental import pallas as pl
from jax.experimental.pallas import tpu as pltpu
from jax.experimental.pallas import tpu_sc as plsc

N = 10000
D = 128

_NC = 2
_NS = 16
_NW = _NC * _NS
_CHUNK = 128
_CPW = 3
_BPW = _CHUNK * _CPW
_B_PAD = _NW * _BPW


def _make_sc_gather():
    mesh = plsc.VectorSubcoreMesh(core_axis_name="c", subcore_axis_name="s")

    @functools.partial(
        pl.kernel,
        mesh=mesh,
        out_type=jax.ShapeDtypeStruct((_B_PAD, D), jnp.float32),
        scratch_types=[
            pltpu.VMEM((_BPW,), jnp.int32),
            pltpu.VMEM((_BPW, D), jnp.float32),
            pltpu.SemaphoreType.DMA,
        ],
    )
    def gather_kernel(table_hbm, idx_hbm, out_hbm, idx_v, rows_v, sem):
        wid = lax.axis_index("s") * _NC + lax.axis_index("c")
        base = wid * _BPW
        pltpu.sync_copy(idx_hbm.at[pl.ds(base, _BPW)], idx_v)
        pltpu.async_copy(table_hbm.at[idx_v], rows_v, sem).wait()
        pltpu.sync_copy(rows_v, out_hbm.at[pl.ds(base, _BPW)])

    return gather_kernel


_sc_gather_cache = []


def _sc_gather(table, idx):
    if not _sc_gather_cache:
        _sc_gather_cache.append(_make_sc_gather())
    return _sc_gather_cache[0](table, idx)


def _prep_body(ue_ref, w1_ref, w2_ref, b2_ref, lw1_ref, lb1_ref, lw2_ref,
               lb2_ref, s1_ref, v_ref, c_ref):
    s1_ref[...] = jnp.dot(ue_ref[...], w1_ref[...],
                          preferred_element_type=jnp.float32)
    t = jnp.dot(lw1_ref[...], lw2_ref[...],
                preferred_element_type=jnp.float32)
    v_ref[...] = jnp.dot(w2_ref[...], t,
                         preferred_element_type=jnp.float32)
    c_ref[...] = (jnp.dot(b2_ref[...], t,
                          preferred_element_type=jnp.float32)
                  + jnp.dot(lb1_ref[...], lw2_ref[...],
                            preferred_element_type=jnp.float32)
                  + lb2_ref[...])


def _prep(user_emb, W1, W2, b2, lw1, lb1, lw2, lb2):
    return pl.pallas_call(
        _prep_body,
        out_shape=(
            jax.ShapeDtypeStruct((N, D), jnp.float32),
            jax.ShapeDtypeStruct((D, 1), jnp.float32),
            jax.ShapeDtypeStruct((1, 1), jnp.float32),
        ),
    )(user_emb, W1, W2, b2.reshape(1, D), lw1, lb1.reshape(1, 16), lw2,
      lb2.reshape(1, 1))


_BM = 400
_NBLK = N // _BM


def _pass_a_body(adj_ref, s1_ref, b1_ref, v_ref, out_ref):
    h = jnp.dot(adj_ref[...], s1_ref[...], preferred_element_type=jnp.float32)
    h = jnp.maximum(h + b1_ref[...], 0.0)
    out_ref[...] = jnp.sum(h * v_ref[...], axis=1, keepdims=True)


def _pass_a(adj, s1, b1, v_row):
    return pl.pallas_call(
        _pass_a_body,
        grid=(_NBLK,),
        in_specs=[
            pl.BlockSpec((_BM, N), lambda m: (m, 0)),
            pl.BlockSpec((N, D), lambda m: (0, 0)),
            pl.BlockSpec((1, D), lambda m: (0, 0)),
            pl.BlockSpec((1, D), lambda m: (0, 0)),
        ],
        out_specs=pl.BlockSpec((_BM, 1), lambda m: (m, 0)),
        out_shape=jax.ShapeDtypeStruct((N, 1), jnp.float32),
        compiler_params=pltpu.CompilerParams(
            dimension_semantics=("arbitrary",)),
    )(adj, s1, b1.reshape(1, D), v_row)


def _pass_b_body(adj_ref, hv_ref, c_ref, out_ref):
    out_ref[...] = (jnp.sum(adj_ref[...] * hv_ref[...], axis=1,
                            keepdims=True) + c_ref[0, 0])


def _pass_b(adj, hv_row, c):
    return pl.pallas_call(
        _pass_b_body,
        grid=(_NBLK,),
        in_specs=[
            pl.BlockSpec((_BM, N), lambda m: (m, 0)),
            pl.BlockSpec((1, N), lambda m: (0, 0)),
            pl.BlockSpec((1, 1), lambda m: (0, 0)),
        ],
        out_specs=pl.BlockSpec((_BM, 1), lambda m: (m, 0)),
        out_shape=jax.ShapeDtypeStruct((N, 1), jnp.float32),
        compiler_params=pltpu.CompilerParams(
            dimension_semantics=("arbitrary",)),
    )(adj, hv_row, c)


def kernel(features, adj, emb_table, W1, b1, W2, b2, lw1, lb1, lw2, lb2):
    idx = jnp.zeros((_B_PAD,), jnp.int32).at[:N].set(
        features.astype(jnp.int32))
    emb_pad = _sc_gather(emb_table, idx)
    user_emb = emb_pad[:N]
    s1, v, c = _prep(user_emb, W1, W2, b2, lw1, lb1, lw2, lb2)
    hv = _pass_a(adj, s1, b1, v.reshape(1, D))
    x = _pass_b(adj, hv.reshape(1, N), c)
    return (x, user_emb)


# speedup vs baseline: 1.3168x; 1.3132x over previous
import functools

import jax
import jax.numpy as jnp
from jax import lax
from jax.experimental import pallas as pl
from jax.experimental.pallas import tpu as pltpu
from jax.experimental.pallas import tpu_sc as plsc

N = 10000
D = 128

_NC = 2
_NS = 16
_NW = _NC * _NS
_CHUNK = 128
_CPW = 3
_BPW = _CHUNK * _CPW
_B_PAD = _NW * _BPW


def _make_sc_gather():
    mesh = plsc.VectorSubcoreMesh(core_axis_name="c", subcore_axis_name="s")

    @functools.partial(
        pl.kernel,
        mesh=mesh,
        out_type=jax.ShapeDtypeStruct((_B_PAD, D), jnp.float32),
        scratch_types=[
            pltpu.VMEM((_BPW,), jnp.int32),
            pltpu.VMEM((_BPW, D), jnp.float32),
            pltpu.SemaphoreType.DMA,
        ],
    )
    def gather_kernel(table_hbm, idx_hbm, out_hbm, idx_v, rows_v, sem):
        wid = lax.axis_index("s") * _NC + lax.axis_index("c")
        base = wid * _BPW
        pltpu.sync_copy(idx_hbm.at[pl.ds(base, _BPW)], idx_v)
        pltpu.async_copy(table_hbm.at[idx_v], rows_v, sem).wait()
        pltpu.sync_copy(rows_v, out_hbm.at[pl.ds(base, _BPW)])

    return gather_kernel


_sc_gather_cache = []


def _sc_gather(table, idx):
    if not _sc_gather_cache:
        _sc_gather_cache.append(_make_sc_gather())
    return _sc_gather_cache[0](table, idx)


def _prep_body(ue_ref, w1_ref, w2_ref, b2_ref, lw1_ref, lb1_ref, lw2_ref,
               lb2_ref, s1_ref, v_ref, c_ref):
    s1_ref[...] = jnp.dot(ue_ref[...], w1_ref[...],
                          preferred_element_type=jnp.float32)
    t = jnp.dot(lw1_ref[...], lw2_ref[...],
                preferred_element_type=jnp.float32)
    v_ref[...] = jnp.dot(w2_ref[...], t,
                         preferred_element_type=jnp.float32)
    c_ref[...] = (jnp.dot(b2_ref[...], t,
                          preferred_element_type=jnp.float32)
                  + jnp.dot(lb1_ref[...], lw2_ref[...],
                            preferred_element_type=jnp.float32)
                  + lb2_ref[...])


def _prep(user_emb, W1, W2, b2, lw1, lb1, lw2, lb2):
    return pl.pallas_call(
        _prep_body,
        out_shape=(
            jax.ShapeDtypeStruct((N, D), jnp.float32),
            jax.ShapeDtypeStruct((D, 1), jnp.float32),
            jax.ShapeDtypeStruct((1, 1), jnp.float32),
        ),
    )(user_emb, W1, W2, b2.reshape(1, D), lw1, lb1.reshape(1, 16), lw2,
      lb2.reshape(1, 1))


_BM = 400
_NBLK = N // _BM


def _pass_a_body(adj_ref, s1_ref, b1_ref, v_ref, out_ref):
    h = jnp.dot(adj_ref[...], s1_ref[...], preferred_element_type=jnp.float32)
    h = jnp.maximum(h + b1_ref[...], 0.0)
    out_ref[...] = jnp.sum(h * v_ref[...], axis=1, keepdims=True)


def _pass_a(adj, s1, b1, v_row):
    return pl.pallas_call(
        _pass_a_body,
        grid=(_NBLK,),
        in_specs=[
            pl.BlockSpec((_BM, N), lambda m: (m, 0)),
            pl.BlockSpec((N, D), lambda m: (0, 0)),
            pl.BlockSpec((1, D), lambda m: (0, 0)),
            pl.BlockSpec((1, D), lambda m: (0, 0)),
        ],
        out_specs=pl.BlockSpec((_BM, 1), lambda m: (m, 0)),
        out_shape=jax.ShapeDtypeStruct((N, 1), jnp.float32),
        compiler_params=pltpu.CompilerParams(
            dimension_semantics=("arbitrary",)),
    )(adj, s1, b1.reshape(1, D), v_row)


def _pass_b_body(adj_ref, hv_ref, c_ref, out_ref):
    out_ref[...] = (jnp.sum(adj_ref[...] * hv_ref[...], axis=1,
                            keepdims=True) + c_ref[0, 0])


def _pass_b(adj, hv_row, c):
    return pl.pallas_call(
        _pass_b_body,
        grid=(_NBLK,),
        in_specs=[
            pl.BlockSpec((_BM, N), lambda m: (m, 0)),
            pl.BlockSpec((1, N), lambda m: (0, 0)),
            pl.BlockSpec((1, 1), lambda m: (0, 0)),
        ],
        out_specs=pl.BlockSpec((_BM, 1), lambda m: (m, 0)),
        out_shape=jax.ShapeDtypeStruct((N, 1), jnp.float32),
        compiler_params=pltpu.CompilerParams(
            dimension_semantics=("arbitrary",)),
    )(adj, hv_row, c)


def kernel(features, adj, emb_table, W1, b1, W2, b2, lw1, lb1, lw2, lb2):
    pad = jnp.arange(N, _B_PAD, dtype=jnp.int32) % jnp.int32(1024)
    idx = jnp.concatenate([features.astype(jnp.int32), pad])
    emb_pad = _sc_gather(emb_table, idx)
    user_emb = emb_pad[:N]
    s1, v, c = _prep(user_emb, W1, W2, b2, lw1, lb1, lw2, lb2)
    hv = _pass_a(adj, s1, b1, v.reshape(1, D))
    x = _pass_b(adj, hv.reshape(1, N), c)
    return (x, user_emb)


# speedup vs baseline: 1.3412x; 1.0185x over previous
import functools

import jax
import jax.numpy as jnp
from jax import lax
from jax.experimental import pallas as pl
from jax.experimental.pallas import tpu as pltpu
from jax.experimental.pallas import tpu_sc as plsc

N = 10000
D = 128

_NC = 2
_NS = 16
_NW = _NC * _NS
_BPW = 384
_B_PAD = _NW * _BPW


def _make_sc_gather():
    mesh = plsc.VectorSubcoreMesh(core_axis_name="c", subcore_axis_name="s")

    @functools.partial(
        pl.kernel,
        mesh=mesh,
        out_type=jax.ShapeDtypeStruct((_B_PAD, D), jnp.float32),
        scratch_types=[
            pltpu.VMEM((_BPW,), jnp.int32),
            pltpu.VMEM((_BPW, D), jnp.float32),
            pltpu.SemaphoreType.DMA,
        ],
    )
    def gather_kernel(table_hbm, idx_hbm, out_hbm, idx_v, rows_v, sem):
        wid = lax.axis_index("s") * _NC + lax.axis_index("c")
        base = wid * _BPW
        pltpu.sync_copy(idx_hbm.at[pl.ds(base, _BPW)], idx_v)
        pltpu.async_copy(table_hbm.at[idx_v], rows_v, sem).wait()
        pltpu.sync_copy(rows_v, out_hbm.at[pl.ds(base, _BPW)])

    return gather_kernel


_sc_gather_cache = []


def _sc_gather(table, idx):
    if not _sc_gather_cache:
        _sc_gather_cache.append(_make_sc_gather())
    return _sc_gather_cache[0](table, idx)


_BM = 200
_NBLK = N // _BM


def _main_body(adj_ref, ue_ref, w1_ref, b1_ref, w2_ref, b2_ref,
               lw1_ref, lb1_ref, lw2_ref, lb2_ref, out_ref,
               st_ref, h_ref):
    p = pl.program_id(0)
    m = pl.program_id(1)

    @pl.when(jnp.logical_and(p == 0, m == 0))
    def _():
        st_ref[...] = jnp.dot(ue_ref[...], w1_ref[...],
                              preferred_element_type=jnp.float32)

    @pl.when(p == 0)
    def _():
        h = jnp.dot(adj_ref[...], st_ref[...],
                    preferred_element_type=jnp.float32)
        h_ref[pl.ds(m * _BM, _BM), :] = jnp.maximum(h + b1_ref[...], 0.0)

    @pl.when(jnp.logical_and(p == 1, m == 0))
    def _():
        st_ref[...] = jnp.dot(h_ref[...], w2_ref[...],
                              preferred_element_type=jnp.float32)

    @pl.when(p == 1)
    def _():
        h2 = jnp.dot(adj_ref[...], st_ref[...],
                     preferred_element_type=jnp.float32) + b2_ref[...]
        y = jnp.dot(h2, lw1_ref[...],
                    preferred_element_type=jnp.float32) + lb1_ref[...]
        out_ref[...] = jnp.dot(y, lw2_ref[...],
                               preferred_element_type=jnp.float32) + lb2_ref[...]


def _main(adj, user_emb, W1, b1, W2, b2, lw1, lb1, lw2, lb2):
    return pl.pallas_call(
        _main_body,
        grid=(2, _NBLK),
        in_specs=[
            pl.BlockSpec((_BM, N), lambda p, m: (m, 0)),
            pl.BlockSpec((N, D), lambda p, m: (0, 0)),
            pl.BlockSpec((D, D), lambda p, m: (0, 0)),
            pl.BlockSpec((1, D), lambda p, m: (0, 0)),
            pl.BlockSpec((D, D), lambda p, m: (0, 0)),
            pl.BlockSpec((1, D), lambda p, m: (0, 0)),
            pl.BlockSpec((D, 16), lambda p, m: (0, 0)),
            pl.BlockSpec((1, 16), lambda p, m: (0, 0)),
            pl.BlockSpec((16, 1), lambda p, m: (0, 0)),
            pl.BlockSpec((1, 1), lambda p, m: (0, 0)),
        ],
        out_specs=pl.BlockSpec((_BM, 1), lambda p, m: (m, 0)),
        out_shape=jax.ShapeDtypeStruct((N, 1), jnp.float32),
        scratch_shapes=[
            pltpu.VMEM((N, D), jnp.float32),
            pltpu.VMEM((N, D), jnp.float32),
        ],
        compiler_params=pltpu.CompilerParams(
            dimension_semantics=("arbitrary", "arbitrary")),
    )(adj, user_emb, W1, b1.reshape(1, D), W2, b2.reshape(1, D),
      lw1, lb1.reshape(1, 16), lw2, lb2.reshape(1, 1))


def kernel(features, adj, emb_table, W1, b1, W2, b2, lw1, lb1, lw2, lb2):
    pad = jnp.arange(N, _B_PAD, dtype=jnp.int32) % jnp.int32(1024)
    idx = jnp.concatenate([features.astype(jnp.int32), pad])
    emb_pad = _sc_gather(emb_table, idx)
    user_emb = emb_pad[:N]
    x = _main(adj, user_emb, W1, b1, W2, b2, lw1, lb1, lw2, lb2)
    return (x, user_emb)


# speedup vs baseline: 1.3875x; 1.0345x over previous
import functools

import jax
import jax.numpy as jnp
from jax import lax
from jax.experimental import pallas as pl
from jax.experimental.pallas import tpu as pltpu
from jax.experimental.pallas import tpu_sc as plsc

N = 10000
D = 128

_NC = 2
_NS = 16
_NW = _NC * _NS
_BPW = 384
_B_PAD = _NW * _BPW


def _make_sc_gather():
    mesh = plsc.VectorSubcoreMesh(core_axis_name="c", subcore_axis_name="s")

    @functools.partial(
        pl.kernel,
        mesh=mesh,
        out_type=jax.ShapeDtypeStruct((_B_PAD, D), jnp.float32),
        scratch_types=[
            pltpu.VMEM((_BPW,), jnp.int32),
            pltpu.VMEM((_BPW, D), jnp.float32),
            pltpu.SemaphoreType.DMA,
        ],
    )
    def gather_kernel(table_hbm, idx_hbm, out_hbm, idx_v, rows_v, sem):
        wid = lax.axis_index("s") * _NC + lax.axis_index("c")
        base = wid * _BPW
        pltpu.sync_copy(idx_hbm.at[pl.ds(base, _BPW)], idx_v)
        pltpu.async_copy(table_hbm.at[idx_v], rows_v, sem).wait()
        pltpu.sync_copy(rows_v, out_hbm.at[pl.ds(base, _BPW)])

    return gather_kernel


_sc_gather_cache = []


def _sc_gather(table, idx):
    if not _sc_gather_cache:
        _sc_gather_cache.append(_make_sc_gather())
    return _sc_gather_cache[0](table, idx)


_BM = 400
_NBLK = N // _BM


def _main_body(adj_ref, ue_ref, w1_ref, b1_ref, w2_ref, b2_ref,
               lw1_ref, lb1_ref, lw2_ref, lb2_ref, out_ref,
               st_ref, h_ref):
    p = pl.program_id(0)
    m = pl.program_id(1)

    @pl.when(jnp.logical_and(p == 0, m == 0))
    def _():
        st_ref[...] = jnp.dot(ue_ref[...], w1_ref[...],
                              preferred_element_type=jnp.float32)

    @pl.when(p == 0)
    def _():
        h = jnp.dot(adj_ref[...], st_ref[...],
                    preferred_element_type=jnp.float32)
        h_ref[pl.ds(m * _BM, _BM), :] = jnp.maximum(h + b1_ref[...], 0.0)

    @pl.when(jnp.logical_and(p == 1, m == 0))
    def _():
        st_ref[...] = jnp.dot(h_ref[...], w2_ref[...],
                              preferred_element_type=jnp.float32)

    @pl.when(p == 1)
    def _():
        h2 = jnp.dot(adj_ref[...], st_ref[...],
                     preferred_element_type=jnp.float32) + b2_ref[...]
        y = jnp.dot(h2, lw1_ref[...],
                    preferred_element_type=jnp.float32) + lb1_ref[...]
        out_ref[...] = jnp.dot(y, lw2_ref[...],
                               preferred_element_type=jnp.float32) + lb2_ref[...]


def _main(adj, user_emb, W1, b1, W2, b2, lw1, lb1, lw2, lb2):
    return pl.pallas_call(
        _main_body,
        grid=(2, _NBLK),
        in_specs=[
            pl.BlockSpec((_BM, N), lambda p, m: (m, 0)),
            pl.BlockSpec((N, D), lambda p, m: (0, 0)),
            pl.BlockSpec((D, D), lambda p, m: (0, 0)),
            pl.BlockSpec((1, D), lambda p, m: (0, 0)),
            pl.BlockSpec((D, D), lambda p, m: (0, 0)),
            pl.BlockSpec((1, D), lambda p, m: (0, 0)),
            pl.BlockSpec((D, 16), lambda p, m: (0, 0)),
            pl.BlockSpec((1, 16), lambda p, m: (0, 0)),
            pl.BlockSpec((16, 1), lambda p, m: (0, 0)),
            pl.BlockSpec((1, 1), lambda p, m: (0, 0)),
        ],
        out_specs=pl.BlockSpec((_BM, 1), lambda p, m: (m, 0)),
        out_shape=jax.ShapeDtypeStruct((N, 1), jnp.float32),
        scratch_shapes=[
            pltpu.VMEM((N, D), jnp.float32),
            pltpu.VMEM((N, D), jnp.float32),
        ],
        compiler_params=pltpu.CompilerParams(
            dimension_semantics=("arbitrary", "arbitrary")),
    )(adj, user_emb, W1, b1.reshape(1, D), W2, b2.reshape(1, D),
      lw1, lb1.reshape(1, 16), lw2, lb2.reshape(1, 1))


def kernel(features, adj, emb_table, W1, b1, W2, b2, lw1, lb1, lw2, lb2):
    pad = jnp.arange(N, _B_PAD, dtype=jnp.int32) % jnp.int32(1024)
    idx = jnp.concatenate([features.astype(jnp.int32), pad])
    emb_pad = _sc_gather(emb_table, idx)
    user_emb = emb_pad[:N]
    x = _main(adj, user_emb, W1, b1, W2, b2, lw1, lb1, lw2, lb2)
    return (x, user_emb)


# speedup vs baseline: 1.4114x; 1.0172x over previous
import functools

import jax
import jax.numpy as jnp
from jax import lax
from jax.experimental import pallas as pl
from jax.experimental.pallas import tpu as pltpu
from jax.experimental.pallas import tpu_sc as plsc

N = 10000
D = 128

_NC = 2
_NS = 16
_NW = _NC * _NS
_BPW = 320
_B_PAD = _NW * _BPW
_WCH = 80


def _make_sc_gather():
    mesh = plsc.VectorSubcoreMesh(core_axis_name="c", subcore_axis_name="s")

    @functools.partial(
        pl.kernel,
        mesh=mesh,
        out_type=jax.ShapeDtypeStruct((N, D), jnp.float32),
        scratch_types=[
            pltpu.VMEM((_BPW,), jnp.int32),
            pltpu.VMEM((_BPW, D), jnp.float32),
            pltpu.SemaphoreType.DMA,
        ],
    )
    def gather_kernel(table_hbm, idx_hbm, out_hbm, idx_v, rows_v, sem):
        wid = lax.axis_index("s") * _NC + lax.axis_index("c")
        base = wid * _BPW
        pltpu.sync_copy(idx_hbm.at[pl.ds(base, _BPW)], idx_v)
        pltpu.async_copy(table_hbm.at[idx_v], rows_v, sem).wait()
        for j in range(_BPW // _WCH):
            lo = base + j * _WCH

            @pl.when(lo + _WCH <= N)
            def _():
                pltpu.sync_copy(rows_v.at[pl.ds(j * _WCH, _WCH)],
                                out_hbm.at[pl.ds(lo, _WCH)])

    return gather_kernel


_sc_gather_cache = []


def _sc_gather(table, idx):
    if not _sc_gather_cache:
        _sc_gather_cache.append(_make_sc_gather())
    return _sc_gather_cache[0](table, idx)


_BM = 400
_NBLK = N // _BM


def _main_body(adj_ref, ue_ref, w1_ref, b1_ref, w2_ref, b2_ref,
               lw1_ref, lb1_ref, lw2_ref, lb2_ref, out_ref,
               st_ref, h_ref):
    p = pl.program_id(0)
    m = pl.program_id(1)

    @pl.when(jnp.logical_and(p == 0, m == 0))
    def _():
        st_ref[...] = jnp.dot(ue_ref[...], w1_ref[...],
                              preferred_element_type=jnp.float32)

    @pl.when(p == 0)
    def _():
        h = jnp.dot(adj_ref[...], st_ref[...],
                    preferred_element_type=jnp.float32)
        h_ref[pl.ds(m * _BM, _BM), :] = jnp.maximum(h + b1_ref[...], 0.0)

    @pl.when(jnp.logical_and(p == 1, m == 0))
    def _():
        st_ref[...] = jnp.dot(h_ref[...], w2_ref[...],
                              preferred_element_type=jnp.float32)

    @pl.when(p == 1)
    def _():
        h2 = jnp.dot(adj_ref[...], st_ref[...],
                     preferred_element_type=jnp.float32) + b2_ref[...]
        y = jnp.dot(h2, lw1_ref[...],
                    preferred_element_type=jnp.float32) + lb1_ref[...]
        out_ref[...] = jnp.dot(y, lw2_ref[...],
                               preferred_element_type=jnp.float32) + lb2_ref[...]


def _main(adj, user_emb, W1, b1, W2, b2, lw1, lb1, lw2, lb2):
    return pl.pallas_call(
        _main_body,
        grid=(2, _NBLK),
        in_specs=[
            pl.BlockSpec((_BM, N), lambda p, m: (m, 0)),
            pl.BlockSpec((N, D), lambda p, m: (0, 0)),
            pl.BlockSpec((D, D), lambda p, m: (0, 0)),
            pl.BlockSpec((1, D), lambda p, m: (0, 0)),
            pl.BlockSpec((D, D), lambda p, m: (0, 0)),
            pl.BlockSpec((1, D), lambda p, m: (0, 0)),
            pl.BlockSpec((D, 16), lambda p, m: (0, 0)),
            pl.BlockSpec((1, 16), lambda p, m: (0, 0)),
            pl.BlockSpec((16, 1), lambda p, m: (0, 0)),
            pl.BlockSpec((1, 1), lambda p, m: (0, 0)),
        ],
        out_specs=pl.BlockSpec((_BM, 1), lambda p, m: (m, 0)),
        out_shape=jax.ShapeDtypeStruct((N, 1), jnp.float32),
        scratch_shapes=[
            pltpu.VMEM((N, D), jnp.float32),
            pltpu.VMEM((N, D), jnp.float32),
        ],
        compiler_params=pltpu.CompilerParams(
            dimension_semantics=("arbitrary", "arbitrary")),
    )(adj, user_emb, W1, b1.reshape(1, D), W2, b2.reshape(1, D),
      lw1, lb1.reshape(1, 16), lw2, lb2.reshape(1, 1))


def kernel(features, adj, emb_table, W1, b1, W2, b2, lw1, lb1, lw2, lb2):
    pad = jnp.arange(N, _B_PAD, dtype=jnp.int32) % jnp.int32(1024)
    idx = jnp.concatenate([features.astype(jnp.int32), pad])
    user_emb = _sc_gather(emb_table, idx)
    x = _main(adj, user_emb, W1, b1, W2, b2, lw1, lb1, lw2, lb2)
    return (x, user_emb)


# speedup vs baseline: 1.4145x; 1.0023x over previous
import functools

import jax
import jax.numpy as jnp
from jax import lax
from jax.experimental import pallas as pl
from jax.experimental.pallas import tpu as pltpu
from jax.experimental.pallas import tpu_sc as plsc

N = 10000
D = 128

_NC = 2
_NS = 16
_NW = _NC * _NS
_BPW = 320
_B_PAD = _NW * _BPW
_WCH = 80


def _make_sc_gather():
    mesh = plsc.VectorSubcoreMesh(core_axis_name="c", subcore_axis_name="s")

    @functools.partial(
        pl.kernel,
        mesh=mesh,
        out_type=jax.ShapeDtypeStruct((N, D), jnp.float32),
        scratch_types=[
            pltpu.VMEM((_BPW,), jnp.int32),
            pltpu.VMEM((_BPW, D), jnp.float32),
            pltpu.SemaphoreType.DMA,
        ],
    )
    def gather_kernel(table_hbm, idx_hbm, out_hbm, idx_v, rows_v, sem):
        wid = lax.axis_index("s") * _NC + lax.axis_index("c")
        base = wid * _BPW
        pltpu.sync_copy(idx_hbm.at[pl.ds(base, _BPW)], idx_v)
        pltpu.async_copy(table_hbm.at[idx_v], rows_v, sem).wait()
        for j in range(_BPW // _WCH):
            lo = base + j * _WCH

            @pl.when(lo + _WCH <= N)
            def _():
                pltpu.sync_copy(rows_v.at[pl.ds(j * _WCH, _WCH)],
                                out_hbm.at[pl.ds(lo, _WCH)])

    return gather_kernel


_sc_gather_cache = []


def _sc_gather(table, idx):
    if not _sc_gather_cache:
        _sc_gather_cache.append(_make_sc_gather())
    return _sc_gather_cache[0](table, idx)


_BM = 400
_NBLK = N // _BM


def _main_body(adj_ref, ue_ref, w1_ref, b1_ref, w2_ref, b2_ref,
               lw1_ref, lb1_ref, lw2_ref, lb2_ref, out_ref,
               st_ref, h_ref):
    p = pl.program_id(0)
    m = pl.program_id(1)

    @pl.when(jnp.logical_and(p == 0, m == 0))
    def _():
        st_ref[...] = jnp.dot(ue_ref[...], w1_ref[...],
                              preferred_element_type=jnp.float32)

    @pl.when(p == 0)
    def _():
        h = jnp.dot(adj_ref[...], st_ref[...],
                    preferred_element_type=jnp.float32)
        h_ref[pl.ds(m * _BM, _BM), :] = jnp.maximum(h + b1_ref[...], 0.0)

    @pl.when(jnp.logical_and(p == 1, m == 0))
    def _():
        st_ref[...] = jnp.dot(h_ref[...], w2_ref[...],
                              preferred_element_type=jnp.float32)

    @pl.when(p == 1)
    def _():
        h2 = jnp.dot(adj_ref[...], st_ref[...],
                     preferred_element_type=jnp.float32) + b2_ref[...]
        y = jnp.dot(h2, lw1_ref[...],
                    preferred_element_type=jnp.float32) + lb1_ref[...]
        out_ref[...] = jnp.dot(y, lw2_ref[...],
                               preferred_element_type=jnp.float32) + lb2_ref[...]


def _main(adj, user_emb, W1, b1, W2, b2, lw1, lb1, lw2, lb2):
    return pl.pallas_call(
        _main_body,
        grid=(2, _NBLK),
        in_specs=[
            pl.BlockSpec((_BM, N),
                         lambda p, m: ((1 - p) * m + p * (_NBLK - 1 - m), 0)),
            pl.BlockSpec((N, D), lambda p, m: (0, 0)),
            pl.BlockSpec((D, D), lambda p, m: (0, 0)),
            pl.BlockSpec((1, D), lambda p, m: (0, 0)),
            pl.BlockSpec((D, D), lambda p, m: (0, 0)),
            pl.BlockSpec((1, D), lambda p, m: (0, 0)),
            pl.BlockSpec((D, 16), lambda p, m: (0, 0)),
            pl.BlockSpec((1, 16), lambda p, m: (0, 0)),
            pl.BlockSpec((16, 1), lambda p, m: (0, 0)),
            pl.BlockSpec((1, 1), lambda p, m: (0, 0)),
        ],
        out_specs=pl.BlockSpec(
            (_BM, 1), lambda p, m: ((1 - p) * m + p * (_NBLK - 1 - m), 0)),
        out_shape=jax.ShapeDtypeStruct((N, 1), jnp.float32),
        scratch_shapes=[
            pltpu.VMEM((N, D), jnp.float32),
            pltpu.VMEM((N, D), jnp.float32),
        ],
        compiler_params=pltpu.CompilerParams(
            dimension_semantics=("arbitrary", "arbitrary")),
    )(adj, user_emb, W1, b1.reshape(1, D), W2, b2.reshape(1, D),
      lw1, lb1.reshape(1, 16), lw2, lb2.reshape(1, 1))


def kernel(features, adj, emb_table, W1, b1, W2, b2, lw1, lb1, lw2, lb2):
    pad = jnp.arange(N, _B_PAD, dtype=jnp.int32) % jnp.int32(1024)
    idx = jnp.concatenate([features.astype(jnp.int32), pad])
    user_emb = _sc_gather(emb_table, idx)
    x = _main(adj, user_emb, W1, b1, W2, b2, lw1, lb1, lw2, lb2)
    return (x, user_emb)
